# baseline (speedup 1.0000x reference)
import jax
import jax.numpy as jnp
from jax import lax
from jax.experimental import pallas as pl
from jax.experimental.pallas import tpu as pltpu
from jax.experimental.pallas import tpu_sc as plsc

_B = 16384
_D = 32
_NT = 4
_V = 100000
_NC = 2
_NS = 16
_NW = _NC * _NS
_BPW = _B // _NW
_L = 16
_NCHUNK = _BPW // _L


def _body(idx_hbm, t4, out_hbm, idx_v, bufs, gsems, ssem):
    wid = lax.axis_index("s") * _NC + lax.axis_index("c")
    base = wid * _BPW
    pltpu.sync_copy(idx_hbm.at[pl.ds(base, _BPW)], idx_v)

    def chunk_gathers(i, _):
        idxvec = idx_v[pl.ds(i * _L, _L)]
        for t in range(_NT):
            pltpu.async_copy(
                t4.at[idxvec + jnp.int32(t * _V)],
                bufs.at[t, pl.ds(i * _L, _L), :],
                gsems.at[t],
            )
        return ()

    lax.fori_loop(0, _NCHUNK, chunk_gathers, (), unroll=False)

    def drain_gathers(t, i):
        idxvec = idx_v[pl.ds(i * _L, _L)]
        pltpu.make_async_copy(
            t4.at[idxvec + jnp.int32(t * _V)],
            bufs.at[t, pl.ds(i * _L, _L), :],
            gsems.at[t],
        ).wait()

    for t in range(_NT):
        lax.fori_loop(
            0, _NCHUNK, lambda i, _, t=t: (drain_gathers(t, i), ())[1], (),
            unroll=False,
        )
        pltpu.async_copy(
            bufs.at[t],
            out_hbm.at[pl.ds(base, _BPW), pl.ds(t * _D, _D)],
            ssem,
        )
    for t in range(_NT):
        pltpu.make_async_copy(
            bufs.at[t],
            out_hbm.at[pl.ds(base, _BPW), pl.ds(t * _D, _D)],
            ssem,
        ).wait()


def kernel(indexes, table0, table1, table2, table3):
    idx = indexes.astype(jnp.int32)
    t4 = jnp.concatenate([table0, table1, table2, table3], axis=0)
    f = pl.kernel(
        _body,
        out_type=jax.ShapeDtypeStruct((_B, _NT * _D), jnp.float32),
        mesh=plsc.VectorSubcoreMesh(core_axis_name="c", subcore_axis_name="s"),
        compiler_params=pltpu.CompilerParams(use_tc_tiling_on_sc=False),
        scratch_types=[
            pltpu.VMEM((_BPW,), jnp.int32),
            pltpu.VMEM((_NT, _BPW, _D), jnp.float32),
            pltpu.SemaphoreType.DMA((_NT,)),
            pltpu.SemaphoreType.DMA,
        ],
    )
    return f(idx, t4)


# speedup vs baseline: 1.6037x; 1.6037x over previous
import jax
import jax.numpy as jnp
from jax import lax
from jax.experimental import pallas as pl
from jax.experimental.pallas import tpu as pltpu
from jax.experimental.pallas import tpu_sc as plsc

_B = 16384
_D = 32
_NT = 4
_V = 100000
_NC = 2
_NS = 16
_NW = _NC * _NS
_BPW = _B // _NW
_L = 16
_R = 64
_NFULL = _V // _R
_TAIL = _V - _NFULL * _R


def _pack_body(t0, t1, t2, t3, big_hbm, bufs, cats, gsem, ssem):
    tables = (t0, t1, t2, t3)
    wid = lax.axis_index("s") * _NC + lax.axis_index("c")

    def fire_reads(c, slot, n):
        for t in range(_NT):
            pltpu.async_copy(
                tables[t].at[pl.ds(c * _R, n), :], bufs.at[t, slot, pl.ds(0, n)],
                gsem,
            )

    def wait_reads(c, slot, n):
        for t in range(_NT):
            pltpu.make_async_copy(
                tables[t].at[pl.ds(c * _R, n), :], bufs.at[t, slot, pl.ds(0, n)],
                gsem,
            ).wait()

    def pack(slot, n):
        for t in range(_NT):
            for r in range(n):
                for h in range(_D // _L):
                    cats[slot, r, pl.ds(t * _D + h * _L, _L)] = (
                        bufs[t, slot, r, pl.ds(h * _L, _L)]
                    )

    def fire_write(c, slot, n):
        pltpu.async_copy(
            cats.at[slot, pl.ds(0, n)], big_hbm.at[pl.ds(c * _R, n), :], ssem
        )

    def wait_write(c, slot, n):
        pltpu.make_async_copy(
            cats.at[slot, pl.ds(0, n)], big_hbm.at[pl.ds(c * _R, n), :], ssem
        ).wait()

    nmine = (_NFULL - wid + _NW - 1) // _NW

    fire_reads(wid, 0, _R)

    def loop_body(k, _):
        c = wid + k * _NW
        slot = lax.rem(k, 2)

        @pl.when(k + 1 < nmine)
        def _():
            fire_reads(c + _NW, 1 - slot, _R)

        @pl.when(k >= 2)
        def _():
            wait_write(c - 2 * _NW, slot, _R)

        wait_reads(c, slot, _R)
        pack(slot, _R)
        fire_write(c, slot, _R)
        return ()

    lax.fori_loop(0, nmine, loop_body, (), unroll=False)

    @pl.when(nmine >= 2)
    def _():
        wait_write(wid + (nmine - 2) * _NW, lax.rem(nmine - 2, 2), _R)

    @pl.when(nmine >= 1)
    def _():
        wait_write(wid + (nmine - 1) * _NW, lax.rem(nmine - 1, 2), _R)

    @pl.when(wid == 0)
    def _():
        fire_reads(_NFULL, 0, _TAIL)
        wait_reads(_NFULL, 0, _TAIL)
        pack(0, _TAIL)
        fire_write(_NFULL, 0, _TAIL)
        wait_write(_NFULL, 0, _TAIL)


def _lookup_body(idx_hbm, big_hbm, out_hbm, idx_v, rows_v, gsem):
    wid = lax.axis_index("s") * _NC + lax.axis_index("c")
    base = wid * _BPW
    pltpu.sync_copy(idx_hbm.at[pl.ds(base, _BPW)], idx_v)

    def chunk_gather(i, _):
        idxvec = idx_v[pl.ds(i * _L, _L)]
        pltpu.async_copy(
            big_hbm.at[idxvec], rows_v.at[pl.ds(i * _L, _L), :], gsem
        )
        return ()

    lax.fori_loop(0, _BPW // _L, chunk_gather, (), unroll=False)

    def chunk_drain(i, _):
        idxvec = idx_v[pl.ds(i * _L, _L)]
        pltpu.make_async_copy(
            big_hbm.at[idxvec], rows_v.at[pl.ds(i * _L, _L), :], gsem
        ).wait()
        return ()

    lax.fori_loop(0, _BPW // _L, chunk_drain, (), unroll=False)
    pltpu.sync_copy(rows_v, out_hbm.at[pl.ds(base, _BPW), :])


def kernel(indexes, table0, table1, table2, table3):
    idx = indexes.astype(jnp.int32)
    mesh = plsc.VectorSubcoreMesh(core_axis_name="c", subcore_axis_name="s")
    pack = pl.kernel(
        _pack_body,
        out_type=jax.ShapeDtypeStruct((_V, _NT * _D), jnp.float32),
        mesh=mesh,
        scratch_types=[
            pltpu.VMEM((_NT, 2, _R, _D), jnp.float32),
            pltpu.VMEM((2, _R, _NT * _D), jnp.float32),
            pltpu.SemaphoreType.DMA,
            pltpu.SemaphoreType.DMA,
        ],
    )
    big = pack(table0, table1, table2, table3)
    lookup = pl.kernel(
        _lookup_body,
        out_type=jax.ShapeDtypeStruct((_B, _NT * _D), jnp.float32),
        mesh=mesh,
        scratch_types=[
            pltpu.VMEM((_BPW,), jnp.int32),
            pltpu.VMEM((_BPW, _NT * _D), jnp.float32),
            pltpu.SemaphoreType.DMA,
        ],
    )
    return lookup(idx, big)


# speedup vs baseline: 1.9801x; 1.2348x over previous
import jax
import jax.numpy as jnp
from jax import lax
from jax.experimental import pallas as pl
from jax.experimental.pallas import tpu as pltpu
from jax.experimental.pallas import tpu_sc as plsc

_B = 16384
_D = 32
_NT = 4
_V = 100000
_NC = 2
_NS = 16
_NW = _NC * _NS
_BPW = _B // _NW
_L = 16
_PCOLS = 512


def _pack_tc_body(t0_ref, t1_ref, t2_ref, t3_ref, out_ref):
    for t, ref in enumerate((t0_ref, t1_ref, t2_ref, t3_ref)):
        out_ref[:, t * _D:(t + 1) * _D] = jnp.transpose(ref[...], (1, 0))


def _lookup_body(idx_hbm, big_hbm, out_hbm, idx_v, rows_v, gsem):
    wid = lax.axis_index("s") * _NC + lax.axis_index("c")
    base = wid * _BPW
    pltpu.sync_copy(idx_hbm.at[pl.ds(base, _BPW)], idx_v)

    def chunk_gather(i, _):
        idxvec = idx_v[pl.ds(i * _L, _L)]
        pltpu.async_copy(
            big_hbm.at[idxvec], rows_v.at[pl.ds(i * _L, _L), :], gsem
        )
        return ()

    lax.fori_loop(0, _BPW // _L, chunk_gather, (), unroll=False)

    def chunk_drain(i, _):
        idxvec = idx_v[pl.ds(i * _L, _L)]
        pltpu.make_async_copy(
            big_hbm.at[idxvec], rows_v.at[pl.ds(i * _L, _L), :], gsem
        ).wait()
        return ()

    lax.fori_loop(0, _BPW // _L, chunk_drain, (), unroll=False)
    pltpu.sync_copy(rows_v, out_hbm.at[pl.ds(base, _BPW), :])


def kernel(indexes, table0, table1, table2, table3):
    idx = indexes.astype(jnp.int32)
    grid = (_V + _PCOLS - 1) // _PCOLS
    in_spec = pl.BlockSpec((_D, _PCOLS), lambda i: (0, i))
    big = pl.pallas_call(
        _pack_tc_body,
        grid=(grid,),
        in_specs=[in_spec] * _NT,
        out_specs=pl.BlockSpec((_PCOLS, _NT * _D), lambda i: (i, 0)),
        out_shape=jax.ShapeDtypeStruct((_V, _NT * _D), jnp.float32),
        compiler_params=pltpu.CompilerParams(
            dimension_semantics=("arbitrary",),
        ),
    )(table0.T, table1.T, table2.T, table3.T)

    lookup = pl.kernel(
        _lookup_body,
        out_type=jax.ShapeDtypeStruct((_B, _NT * _D), jnp.float32),
        mesh=plsc.VectorSubcoreMesh(core_axis_name="c", subcore_axis_name="s"),
        scratch_types=[
            pltpu.VMEM((_BPW,), jnp.int32),
            pltpu.VMEM((_BPW, _NT * _D), jnp.float32),
            pltpu.SemaphoreType.DMA,
        ],
    )
    return lookup(idx, big)
